# baseline (speedup 1.0000x reference)
import jax
import jax.numpy as jnp
from jax import lax
from jax.experimental import pallas as pl
from jax.experimental.pallas import tpu as pltpu
from jax.experimental.pallas import tpu_sc as plsc

B, G, D, N = 16, 4096, 128, 16384
NC, NS, L = 2, 16, 16
PHASES = B // NC
CHUNK = 32
PC = N // NS // CHUNK
NBUF = 2
NFULL = PC // NBUF
TAIL = PC - NFULL * NBUF
SLAB_ROWS = G // NS


def _sc_body(field_hbm, pos_hbm, out_hbm, *scr):
    pos_v = scr[0:NBUF]
    w_v = scr[NBUF:2 * NBUF]
    idx0_v = scr[2 * NBUF:3 * NBUF]
    idx1_v = scr[3 * NBUF:4 * NBUF]
    f0_v = scr[4 * NBUF:5 * NBUF]
    f1_v = scr[5 * NBUF:6 * NBUF]
    o_v = scr[6 * NBUF:7 * NBUF]
    gsem = scr[7 * NBUF:8 * NBUF]
    ssem = scr[8 * NBUF:9 * NBUF]
    slabs = scr[9 * NBUF:9 * NBUF + 2]
    slsem = scr[9 * NBUF + 2:9 * NBUF + 4]

    core = lax.axis_index("c")
    sid = lax.axis_index("s")

    def fire_slab(p, sp):
        b = PHASES * core + p
        pltpu.async_copy(field_hbm.at[b, pl.ds(sid * SLAB_ROWS, SLAB_ROWS)],
                         slabs[sp].at[pl.ds(sid * SLAB_ROWS, SLAB_ROWS)],
                         slsem[sp])

    def wait_slab(sp):
        pltpu.make_async_copy(
            field_hbm.at[0, pl.ds(0, SLAB_ROWS)],
            slabs[sp].at[pl.ds(sid * SLAB_ROWS, SLAB_ROWS)],
            slsem[sp]).wait()

    def fire_gather(fbase, c, par, sp):
        pltpu.sync_copy(pos_hbm.at[pl.ds(fbase + c * CHUNK, CHUNK)],
                        pos_v[par])
        for k in range(CHUNK // L):
            p = pos_v[par][pl.ds(k * L, L)]
            ix = jnp.minimum(jnp.maximum(p * float(G - 1), 0.0),
                             float(G - 1))
            i0 = ix.astype(jnp.int32)
            w = ix - i0.astype(jnp.float32)
            i1 = jnp.minimum(i0 + 1, G - 1)
            idx0_v[par][pl.ds(k * L, L)] = i0
            idx1_v[par][pl.ds(k * L, L)] = i1
            for s in range(L):
                w_v[par][k * L + s, :] = jnp.broadcast_to(w[s], (L,))
        pltpu.async_copy(slabs[sp].at[idx0_v[par]], f0_v[par], gsem[par])
        pltpu.async_copy(slabs[sp].at[idx1_v[par]], f1_v[par], gsem[par])

    def wait_gather(par, sp):
        pltpu.make_async_copy(slabs[sp].at[idx0_v[par]], f0_v[par],
                              gsem[par]).wait()
        pltpu.make_async_copy(slabs[sp].at[idx1_v[par]], f1_v[par],
                              gsem[par]).wait()

    def lerp(par):
        def sample_body(s, carry):
            wb = w_v[par][s, :]
            for j in range(D // L):
                a = f0_v[par][s, pl.ds(j * L, L)]
                b = f1_v[par][s, pl.ds(j * L, L)]
                o_v[par][s, pl.ds(j * L, L)] = a + wb * (b - a)
            return carry

        lax.fori_loop(0, CHUNK, sample_body, 0)

    def fire_scatter(fbase, c, par):
        pltpu.async_copy(o_v[par],
                         out_hbm.at[pl.ds(fbase + c * CHUNK, CHUNK)],
                         ssem[par])

    def wait_scatter(par):
        pltpu.make_async_copy(o_v[par], out_hbm.at[pl.ds(0, CHUNK)],
                              ssem[par]).wait()

    def process_phase(pp, off):
        p = 2 * pp + off
        sp = off
        b = PHASES * core + p
        fbase = b * N + sid * (PC * CHUNK)

        wait_slab(sp)
        plsc.subcore_barrier()
        pl.when(p + 1 < PHASES)(lambda: fire_slab(p + 1, 1 - sp))

        for par in range(NBUF):
            fire_gather(fbase, par, par, sp)

        def rot_body(cc, carry):
            for par in range(NBUF):
                c = NBUF * cc + par
                wait_gather(par, sp)
                pl.when((p > 0) | (cc > 0))(
                    lambda par=par: wait_scatter(par))
                lerp(par)
                fire_scatter(fbase, c, par)
                pl.when(c + NBUF < PC)(
                    lambda c=c, par=par: fire_gather(fbase, c + NBUF, par,
                                                     sp))
            return carry

        lax.fori_loop(0, NFULL, rot_body, 0)

        for t in range(TAIL):
            c = NFULL * NBUF + t
            par = c % NBUF
            wait_gather(par, sp)
            wait_scatter(par)
            lerp(par)
            fire_scatter(fbase, c, par)

    fire_slab(0, 0)

    def phase_pair(pp, carry):
        process_phase(pp, 0)
        process_phase(pp, 1)
        return carry

    lax.fori_loop(0, PHASES // 2, phase_pair, 0)

    for par in range(NBUF):
        wait_scatter(par)


def kernel(field, grid_points, sample_positions):
    del grid_points
    pos_flat = sample_positions.reshape(B * N)
    mesh = plsc.VectorSubcoreMesh(core_axis_name="c", subcore_axis_name="s",
                                  num_cores=NC, num_subcores=NS)
    out2d = pl.kernel(
        _sc_body,
        out_type=jax.ShapeDtypeStruct((B * N, D), jnp.float32),
        mesh=mesh,
        scratch_types=(
            [pltpu.VMEM((CHUNK,), jnp.float32)] * NBUF
            + [pltpu.VMEM((CHUNK, L), jnp.float32)] * NBUF
            + [pltpu.VMEM((CHUNK,), jnp.int32)] * NBUF
            + [pltpu.VMEM((CHUNK,), jnp.int32)] * NBUF
            + [pltpu.VMEM((CHUNK, D), jnp.float32)] * NBUF
            + [pltpu.VMEM((CHUNK, D), jnp.float32)] * NBUF
            + [pltpu.VMEM((CHUNK, D), jnp.float32)] * NBUF
            + [pltpu.SemaphoreType.DMA] * NBUF
            + [pltpu.SemaphoreType.DMA] * NBUF
            + [pltpu.VMEM_SHARED((G, D), jnp.float32)] * 2
            + [pltpu.SemaphoreType.DMA] * 2
        ),
    )(field, pos_flat)
    return out2d.reshape(B, N, D)


# speedup vs baseline: 1.3953x; 1.3953x over previous
import jax
import jax.numpy as jnp
from jax import lax
from jax.experimental import pallas as pl
from jax.experimental.pallas import tpu as pltpu
from jax.experimental.pallas import tpu_sc as plsc

B, G, D, N = 16, 4096, 128, 16384
NC, NS, L = 2, 16, 16
NW = NC * NS
TOTAL = B * N
PER_W = TOTAL // NW
CHUNK = 64
NCHUNK = PER_W // CHUNK
NBUF = 3
NFULL = NCHUNK // NBUF
TAIL = NCHUNK - NFULL * NBUF


def _sc_body(field_hbm, pos_hbm, out_hbm, *scr):
    pos_all = scr[0]
    w_v = scr[NBUF:2 * NBUF]
    idx0_v = scr[2 * NBUF:3 * NBUF]
    idx1_v = scr[3 * NBUF:4 * NBUF]
    f0_v = scr[4 * NBUF:5 * NBUF]
    f1_v = scr[5 * NBUF:6 * NBUF]
    o_v = scr[6 * NBUF:7 * NBUF]
    gsem = scr[7 * NBUF:8 * NBUF]
    ssem = scr[8 * NBUF:9 * NBUF]

    wid = lax.axis_index("s") * NC + lax.axis_index("c")
    wbase = wid * PER_W
    b_off = (wbase // N) * G

    def fire_gather(c, par):
        for k in range(CHUNK // L):
            p = pos_all[pl.ds(c * CHUNK + k * L, L)]
            ix = jnp.minimum(jnp.maximum(p * float(G - 1), 0.0),
                             float(G - 1))
            i0 = ix.astype(jnp.int32)
            w = ix - i0.astype(jnp.float32)
            i1 = jnp.minimum(i0 + 1, G - 1)
            idx0_v[par][pl.ds(k * L, L)] = i0 + b_off
            idx1_v[par][pl.ds(k * L, L)] = i1 + b_off
            for s in range(L):
                w_v[par][k * L + s, :] = jnp.broadcast_to(w[s], (L,))
        pltpu.async_copy(field_hbm.at[idx0_v[par]], f0_v[par], gsem[par])
        pltpu.async_copy(field_hbm.at[idx1_v[par]], f1_v[par], gsem[par])

    def wait_gather(par):
        pltpu.make_async_copy(field_hbm.at[idx0_v[par]], f0_v[par],
                              gsem[par]).wait()
        pltpu.make_async_copy(field_hbm.at[idx1_v[par]], f1_v[par],
                              gsem[par]).wait()

    def lerp(par):
        def sample_body(s, carry):
            wb = w_v[par][s, :]
            for j in range(D // L):
                a = f0_v[par][s, pl.ds(j * L, L)]
                b = f1_v[par][s, pl.ds(j * L, L)]
                o_v[par][s, pl.ds(j * L, L)] = a + wb * (b - a)
            return carry

        lax.fori_loop(0, CHUNK, sample_body, 0)

    def fire_scatter(c, par):
        pltpu.async_copy(o_v[par], out_hbm.at[pl.ds(wbase + c * CHUNK, CHUNK)],
                         ssem[par])

    def wait_scatter(par):
        pltpu.make_async_copy(o_v[par], out_hbm.at[pl.ds(0, CHUNK)],
                              ssem[par]).wait()

    pltpu.sync_copy(pos_hbm.at[pl.ds(wbase, PER_W)], pos_all)
    for par in range(NBUF):
        fire_gather(par, par)

    def rot_body(cc, carry):
        for par in range(NBUF):
            c = NBUF * cc + par
            wait_gather(par)
            pl.when(cc > 0)(lambda par=par: wait_scatter(par))
            lerp(par)
            fire_scatter(c, par)
            pl.when(c + NBUF < NCHUNK)(
                lambda c=c, par=par: fire_gather(c + NBUF, par))
        return carry

    lax.fori_loop(0, NFULL, rot_body, 0)

    for par in range(TAIL):
        c = NFULL * NBUF + par
        wait_gather(par)
        wait_scatter(par)
        lerp(par)
        fire_scatter(c, par)

    for par in range(NBUF):
        wait_scatter(par)


def kernel(field, grid_points, sample_positions):
    del grid_points
    field2d = field.reshape(B * G, D)
    pos_flat = sample_positions.reshape(TOTAL)
    mesh = plsc.VectorSubcoreMesh(core_axis_name="c", subcore_axis_name="s",
                                  num_cores=NC, num_subcores=NS)
    out2d = pl.kernel(
        _sc_body,
        out_type=jax.ShapeDtypeStruct((TOTAL, D), jnp.float32),
        mesh=mesh,
        scratch_types=(
            [pltpu.VMEM((PER_W,), jnp.float32)]
            + [None] * (NBUF - 1)
            + [pltpu.VMEM((CHUNK, L), jnp.float32)] * NBUF
            + [pltpu.VMEM((CHUNK,), jnp.int32)] * NBUF
            + [pltpu.VMEM((CHUNK,), jnp.int32)] * NBUF
            + [pltpu.VMEM((CHUNK, D), jnp.float32)] * NBUF
            + [pltpu.VMEM((CHUNK, D), jnp.float32)] * NBUF
            + [pltpu.VMEM((CHUNK, D), jnp.float32)] * NBUF
            + [pltpu.SemaphoreType.DMA] * NBUF
            + [pltpu.SemaphoreType.DMA] * NBUF
        ),
    )(field2d, pos_flat)
    return out2d.reshape(B, N, D)
